# baseline (speedup 1.0000x reference)
import functools
import math

import numpy as np
import jax
import jax.numpy as jnp
from jax import lax
from jax.experimental import pallas as pl
from jax.experimental.pallas import tpu as pltpu
from jax.experimental.pallas import tpu_sc as plsc

N = 10000
E = 320000
H = 128
LAT = 32
HD = 256
NMAX = 50
DC = 128
NG = 200
AH = NMAX * (NMAX - 1) // 2
AHP = 1280
ADJF = NMAX * NMAX

NC, NS = 2, 16
NW = NC * NS
EPT = E // NW
EB = 80
KB = 128
EPAD = NW * KB * EB
CH = 8
NCH = KB // CH
NBUF = 4
NJUNK = 8
NA = N + NJUNK
RPT = 624
TOFF = NS * RPT
TAIL = N - TOFF

BR = 1000
NBLK = N // BR

_BNS = 1.0 / math.sqrt(1.0 + 1e-5)

_IU = np.triu_indices(NMAX, 1)
_PFULL = np.zeros((AHP, ADJF), np.float32)
_PFULL[np.arange(AH), _IU[0] * NMAX + _IU[1]] = 1.0
_PFULL[np.arange(AH), _IU[1] * NMAX + _IU[0]] = 1.0


def _leaky(t):
    return jnp.where(t > 0, t, 0.2 * t)


def _sc_scatter_body(h_hbm, z_hbm, src_hbm, dst_hbm, out_hbm,
                     src_v, dst_v, bufs, aggs, semsG, semsS, semI):
    buf = list(bufs)
    semG = list(semsG)
    semS = list(semsS)
    agg = aggs
    cid = lax.axis_index("c")
    sid = lax.axis_index("s")
    wid = cid * NS + sid
    row0 = sid * RPT

    @pl.when(cid == 0)
    def _():
        pltpu.sync_copy(h_hbm.at[pl.ds(row0, RPT)], agg.at[pl.ds(row0, RPT)])

        @pl.when(sid == NS - 1)
        def _():
            pltpu.sync_copy(h_hbm.at[pl.ds(TOFF, TAIL)], agg.at[pl.ds(TOFF, TAIL)])

    @pl.when(cid != 0)
    def _():
        pltpu.sync_copy(z_hbm.at[pl.ds(row0, RPT)], agg.at[pl.ds(row0, RPT)])

        @pl.when(sid == NS - 1)
        def _():
            pltpu.sync_copy(z_hbm.at[pl.ds(TOFF, TAIL)], agg.at[pl.ds(TOFF, TAIL)])

    pltpu.sync_copy(src_hbm.at[pl.ds(wid * KB, CH)], src_v.at[0])
    pltpu.sync_copy(dst_hbm.at[pl.ds(wid * KB, CH)], dst_v.at[0])
    pltpu.async_copy(h_hbm.at[src_v.at[0, 0]], buf[0], semG[0])
    pltpu.async_copy(h_hbm.at[src_v.at[0, 1]], buf[1], semG[1])

    plsc.subcore_barrier()

    def chunk(c, s):
        for b in range(CH):
            jb = b % NBUF
            jn = (b + 2) % NBUF

            if b >= 2:
                pltpu.make_async_copy(
                    buf[jn], agg.at[dst_v.at[s, b - 2]], semS[jn]).wait()
            else:
                @pl.when(c > 0)
                def _():
                    pltpu.make_async_copy(
                        buf[jn], agg.at[dst_v.at[1 - s, b + CH - 2]],
                        semS[jn]).wait()

            if b == 2:
                @pl.when(c + 1 < NCH)
                def _():
                    base = wid * KB + (c + 1) * CH
                    pltpu.async_copy(src_hbm.at[pl.ds(base, CH)],
                                     src_v.at[1 - s], semI)
                    pltpu.async_copy(dst_hbm.at[pl.ds(base, CH)],
                                     dst_v.at[1 - s], semI)

            if b == CH - 2:
                @pl.when(c + 1 < NCH)
                def _():
                    pltpu.make_async_copy(src_hbm.at[pl.ds(0, CH)],
                                          src_v.at[1 - s], semI).wait()
                    pltpu.make_async_copy(dst_hbm.at[pl.ds(0, CH)],
                                          dst_v.at[1 - s], semI).wait()

            if b < CH - 2:
                pltpu.async_copy(h_hbm.at[src_v.at[s, b + 2]], buf[jn], semG[jn])
            else:
                @pl.when(c + 1 < NCH)
                def _():
                    pltpu.async_copy(h_hbm.at[src_v.at[1 - s, b - (CH - 2)]],
                                     buf[jn], semG[jn])

            pltpu.make_async_copy(h_hbm.at[src_v.at[s, b]], buf[jb], semG[jb]).wait()
            pltpu.async_copy(buf[jb], agg.at[dst_v.at[s, b]], semS[jb], add=True)

    @pl.loop(0, NCH, step=2)
    def _(ci):
        chunk(ci, 0)
        chunk(ci + 1, 1)

    pltpu.make_async_copy(buf[(CH - 2) % NBUF], agg.at[dst_v.at[1, CH - 2]],
                          semS[(CH - 2) % NBUF]).wait()
    pltpu.make_async_copy(buf[(CH - 1) % NBUF], agg.at[dst_v.at[1, CH - 1]],
                          semS[(CH - 1) % NBUF]).wait()

    plsc.subcore_barrier()

    pltpu.sync_copy(agg.at[pl.ds(row0, RPT)], out_hbm.at[cid, pl.ds(row0, RPT)])

    @pl.when(sid == NS - 1)
    def _():
        pltpu.sync_copy(agg.at[pl.ds(TOFF, TAIL)], out_hbm.at[cid, pl.ds(TOFF, TAIL)])


@functools.lru_cache(maxsize=1)
def _sc_scatter_kernel():
    return pl.kernel(
        _sc_scatter_body,
        out_type=jax.ShapeDtypeStruct((2, N, H), jnp.float32),
        mesh=plsc.VectorSubcoreMesh(core_axis_name="c", subcore_axis_name="s",
                                    num_cores=NC, num_subcores=NS),
        scratch_types=[
            pltpu.VMEM((2, CH, EB), jnp.int32),
            pltpu.VMEM((2, CH, EB), jnp.int32),
            tuple(pltpu.VMEM((EB, H), jnp.float32) for _ in range(NBUF)),
            pltpu.VMEM_SHARED((NA, H), jnp.float32),
            tuple(pltpu.SemaphoreType.DMA for _ in range(NBUF)),
            tuple(pltpu.SemaphoreType.DMA for _ in range(NBUF)),
            pltpu.SemaphoreType.DMA,
        ],
    )


def _sc_scatter(h, zeros, src2, dst2):
    return _sc_scatter_kernel()(h, zeros, src2, dst2)


def _mlp_body(p0_ref, p1_ref, W_ref, b_ref, s_ref, be_ref, W2_ref, b2_ref, o_ref):
    a = p0_ref[...] + p1_ref[...]
    t = jnp.dot(a, W_ref[...], preferred_element_type=jnp.float32) + b_ref[...]
    t = _leaky(t)
    t = t * s_ref[...] + be_ref[...]
    t = jnp.dot(t, W2_ref[...], preferred_element_type=jnp.float32) + b2_ref[...]
    o_ref[...] = _leaky(t)


def _mlp(p0, p1, W, b, s, be, W2, b2):
    full = lambda shp: pl.BlockSpec(shp, lambda i: (0,) * len(shp))
    return pl.pallas_call(
        _mlp_body,
        grid=(NBLK,),
        in_specs=[
            pl.BlockSpec((BR, H), lambda i: (i, 0)),
            pl.BlockSpec((BR, H), lambda i: (i, 0)),
            full((H, H)), full((1, H)), full((1, H)), full((1, H)),
            full((H, H)), full((1, H)),
        ],
        out_specs=pl.BlockSpec((BR, H), lambda i: (i, 0)),
        out_shape=jax.ShapeDtypeStruct((N, H), jnp.float32),
    )(p0, p1, W, b, s, be, W2, b2)


def _mlp_pool_body(p0_ref, p1_ref, batch_ref, W_ref, b_ref, s_ref, be_ref,
                   W2_ref, b2_ref, o_ref):
    a = p0_ref[...] + p1_ref[...]
    t = jnp.dot(a, W_ref[...], preferred_element_type=jnp.float32) + b_ref[...]
    t = _leaky(t)
    t = t * s_ref[...] + be_ref[...]
    t = jnp.dot(t, W2_ref[...], preferred_element_type=jnp.float32) + b2_ref[...]
    t = _leaky(t)
    bb = batch_ref[0]
    onehot = (lax.broadcasted_iota(jnp.int32, (NG, BR), 0) == bb)
    contrib = jnp.dot(onehot.astype(jnp.float32), t,
                      preferred_element_type=jnp.float32)

    @pl.when(pl.program_id(0) == 0)
    def _():
        o_ref[...] = contrib

    @pl.when(pl.program_id(0) != 0)
    def _():
        o_ref[...] += contrib


def _mlp_pool(p0, p1, batch3, W, b, s, be, W2, b2):
    full = lambda shp: pl.BlockSpec(shp, lambda i: (0,) * len(shp))
    return pl.pallas_call(
        _mlp_pool_body,
        grid=(NBLK,),
        in_specs=[
            pl.BlockSpec((BR, H), lambda i: (i, 0)),
            pl.BlockSpec((BR, H), lambda i: (i, 0)),
            pl.BlockSpec((1, 1, BR), lambda i: (i, 0, 0)),
            full((H, H)), full((1, H)), full((1, H)), full((1, H)),
            full((H, H)), full((1, H)),
        ],
        out_specs=pl.BlockSpec((NG, H), lambda i: (0, 0)),
        out_shape=jax.ShapeDtypeStruct((NG, H), jnp.float32),
    )(p0, p1, batch3, W, b, s, be, W2, b2)


def _decoder_body(g_ref, cond_ref, es_ref, eb_ref, fcW_ref, fcb_ref,
                  muW_ref, mub_ref, c0W_ref, c0b_ref, c1W_ref, c1b_ref,
                  d0Wz_ref, d0Wc_ref, d0b_ref, s0_ref, b0_ref,
                  d1Wh_ref, d1Wc_ref, d1b_ref, s1_ref, b1_ref,
                  Wd_ref, gdt_ref, P_ref, o_ref):
    f32 = jnp.float32
    dot = lambda a, b: jnp.dot(a, b, preferred_element_type=f32)
    gb = g_ref[...] * es_ref[...] + eb_ref[...]
    gf = dot(gb, fcW_ref[...]) + fcb_ref[...]
    z = dot(gf, muW_ref[...]) + mub_ref[...]
    c = jnp.maximum(dot(cond_ref[...], c0W_ref[...]) + c0b_ref[...], 0.0)
    c = dot(c, c1W_ref[...]) + c1b_ref[...]
    h0 = jnp.maximum(dot(z, d0Wz_ref[...]) + dot(c, d0Wc_ref[...])
                     + d0b_ref[...], 0.0)
    h0 = h0 * s0_ref[...] + b0_ref[...]
    h1 = jnp.maximum(dot(h0, d1Wh_ref[...]) + dot(c, d1Wc_ref[...])
                     + d1b_ref[...], 0.0)
    h1 = h1 * s1_ref[...] + b1_ref[...]
    delta = dot(h1, Wd_ref[...]) + gdt_ref[...]
    vals = (delta >= 0).astype(jnp.bfloat16)
    o_ref[...] = dot(vals, P_ref[...])


def _decoder(g, cond, es, eb, fcW, fcb, muW, mub, c0W, c0b, c1W, c1b,
             d0Wz, d0Wc, d0b, s0, b0, d1Wh, d1Wc, d1b, s1, b1, Wd, gdt, P):
    return pl.pallas_call(
        _decoder_body,
        out_shape=jax.ShapeDtypeStruct((NG, ADJF), jnp.float32),
    )(g, cond, es, eb, fcW, fcb, muW, mub, c0W, c0b, c1W, c1b,
      d0Wz, d0Wc, d0b, s0, b0, d1Wh, d1Wc, d1b, s1, b1, Wd, gdt, P)


def kernel(x, cond, params, edge_index, batch):
    p = params
    f32 = jnp.float32
    row = lambda v: v.reshape(1, -1).astype(f32)

    npad = EPAD - E
    src2 = jnp.concatenate(
        [edge_index[0], jnp.zeros((npad,), jnp.int32)]).reshape(NW * KB, EB)
    dst2 = jnp.concatenate(
        [edge_index[1],
         N + (jnp.arange(npad, dtype=jnp.int32) & (NJUNK - 1))]).reshape(NW * KB, EB)
    zeros = jnp.zeros((N, H), f32)
    batch3 = batch.reshape(NBLK, 1, BR)

    pp = _sc_scatter(x, zeros, src2, dst2)
    h1 = _mlp(pp[0], pp[1], p['c0W'], row(p['c0b']), row(p['c0g'] * _BNS),
              row(p['c0be']), p['c0W2'], row(p['c0b2']))
    pp2 = _sc_scatter(h1, zeros, src2, dst2)
    g = _mlp_pool(pp2[0], pp2[1], batch3, p['c1W'], row(p['c1b']),
                  row(p['c1g'] * _BNS), row(p['c1be']), p['c1W2'], row(p['c1b2']))

    Wd = p['d2W'][:, 0::2] - p['d2W'][:, 1::2]
    Wd = jnp.pad(Wd, ((0, 0), (0, AHP - AH)))
    gn = jax.random.gumbel(jax.random.key(42), (NG, AH, 2), jnp.float32)
    gdiff = gn[:, :, 0] - gn[:, :, 1] + (p['d2b'][0::2] - p['d2b'][1::2])[None, :]
    gdt = jnp.pad(gdiff, ((0, 0), (0, AHP - AH)), constant_values=-1e9)
    P = jnp.asarray(_PFULL, jnp.bfloat16)

    adjf = _decoder(
        g, cond, row(p['ebn_g'] * _BNS), row(p['ebn_b']),
        p['fcW'], row(p['fcb']), p['muW'], row(p['mub']),
        p['cm0W'], row(p['cm0b']), p['cm1W'], row(p['cm1b']),
        p['d0W'][:LAT], p['d0W'][LAT:], row(p['d0b']),
        row(p['dbn0_g'] * _BNS), row(p['dbn0_b']),
        p['d1W'][:HD], p['d1W'][HD:], row(p['d1b']),
        row(p['dbn1_g'] * _BNS), row(p['dbn1_b']),
        Wd, gdt, P)
    return adjf.reshape(NG, NMAX, NMAX)


# speedup vs baseline: 1.2041x; 1.2041x over previous
import functools
import math

import numpy as np
import jax
import jax.numpy as jnp
from jax import lax
from jax.experimental import pallas as pl
from jax.experimental.pallas import tpu as pltpu
from jax.experimental.pallas import tpu_sc as plsc

N = 10000
E = 320000
H = 128
LAT = 32
HD = 256
NMAX = 50
DC = 128
NG = 200
AH = NMAX * (NMAX - 1) // 2
AHP = 1280
ADJF = NMAX * NMAX

NC, NS = 2, 16
NW = NC * NS
EPT = E // NW
EB = 80
KB = 128
EPAD = NW * KB * EB
CH = 8
NCH = KB // CH
NBUF = 4
NJUNK = 16
NA = N + NJUNK
RPT = 624
TOFF = NS * RPT
TAIL = N - TOFF

BR = 1000
NBLK = N // BR

_BNS = 1.0 / math.sqrt(1.0 + 1e-5)

_IU = np.triu_indices(NMAX, 1)
_PFULL = np.zeros((AHP, ADJF), np.float32)
_PFULL[np.arange(AH), _IU[0] * NMAX + _IU[1]] = 1.0
_PFULL[np.arange(AH), _IU[1] * NMAX + _IU[0]] = 1.0


def _leaky(t):
    return jnp.where(t > 0, t, 0.2 * t)


def _sc_scatter_body(h_hbm, z_hbm, src_hbm, dst_hbm, out_hbm,
                     src_v, dst_v, bufs, aggs, semsG, semsS, semI):
    buf = list(bufs)
    semG = list(semsG)
    semS = list(semsS)
    agg = aggs
    cid = lax.axis_index("c")
    sid = lax.axis_index("s")
    wid = cid * NS + sid
    row0 = sid * RPT

    @pl.when(cid == 0)
    def _():
        pltpu.sync_copy(h_hbm.at[pl.ds(row0, RPT)], agg.at[pl.ds(row0, RPT)])

        @pl.when(sid == NS - 1)
        def _():
            pltpu.sync_copy(h_hbm.at[pl.ds(TOFF, TAIL)], agg.at[pl.ds(TOFF, TAIL)])

    @pl.when(cid != 0)
    def _():
        pltpu.sync_copy(z_hbm.at[pl.ds(row0, RPT)], agg.at[pl.ds(row0, RPT)])

        @pl.when(sid == NS - 1)
        def _():
            pltpu.sync_copy(z_hbm.at[pl.ds(TOFF, TAIL)], agg.at[pl.ds(TOFF, TAIL)])

    pltpu.sync_copy(src_hbm.at[pl.ds(wid * KB, CH)], src_v.at[0])
    pltpu.sync_copy(dst_hbm.at[pl.ds(wid * KB, CH)], dst_v.at[0])
    pltpu.async_copy(h_hbm.at[src_v.at[0, 0]], buf[0], semG[0])
    pltpu.async_copy(h_hbm.at[src_v.at[0, 1]], buf[1], semG[1])

    plsc.subcore_barrier()

    def chunk(c, s):
        for b in range(CH):
            jb = b % NBUF
            jn = (b + 2) % NBUF

            if b >= 2:
                pltpu.make_async_copy(
                    buf[jn], agg.at[dst_v.at[s, b - 2]], semS[jn]).wait()
            else:
                @pl.when(c > 0)
                def _():
                    pltpu.make_async_copy(
                        buf[jn], agg.at[dst_v.at[1 - s, b + CH - 2]],
                        semS[jn]).wait()

            if b == 2:
                @pl.when(c + 1 < NCH)
                def _():
                    base = wid * KB + (c + 1) * CH
                    pltpu.async_copy(src_hbm.at[pl.ds(base, CH)],
                                     src_v.at[1 - s], semI)
                    pltpu.async_copy(dst_hbm.at[pl.ds(base, CH)],
                                     dst_v.at[1 - s], semI)

            if b == CH - 2:
                @pl.when(c + 1 < NCH)
                def _():
                    pltpu.make_async_copy(src_hbm.at[pl.ds(0, CH)],
                                          src_v.at[1 - s], semI).wait()
                    pltpu.make_async_copy(dst_hbm.at[pl.ds(0, CH)],
                                          dst_v.at[1 - s], semI).wait()

            if b < CH - 2:
                pltpu.async_copy(h_hbm.at[src_v.at[s, b + 2]], buf[jn], semG[jn])
            else:
                @pl.when(c + 1 < NCH)
                def _():
                    pltpu.async_copy(h_hbm.at[src_v.at[1 - s, b - (CH - 2)]],
                                     buf[jn], semG[jn])

            pltpu.make_async_copy(h_hbm.at[src_v.at[s, b]], buf[jb], semG[jb]).wait()
            pltpu.async_copy(buf[jb], agg.at[dst_v.at[s, b]], semS[jb], add=True)

    @pl.loop(0, NCH, step=2)
    def _(ci):
        chunk(ci, 0)
        chunk(ci + 1, 1)

    pltpu.make_async_copy(buf[(CH - 2) % NBUF], agg.at[dst_v.at[1, CH - 2]],
                          semS[(CH - 2) % NBUF]).wait()
    pltpu.make_async_copy(buf[(CH - 1) % NBUF], agg.at[dst_v.at[1, CH - 1]],
                          semS[(CH - 1) % NBUF]).wait()

    plsc.subcore_barrier()

    pltpu.sync_copy(agg.at[pl.ds(row0, RPT)], out_hbm.at[cid, pl.ds(row0, RPT)])

    @pl.when(sid == NS - 1)
    def _():
        pltpu.sync_copy(agg.at[pl.ds(TOFF, TAIL)], out_hbm.at[cid, pl.ds(TOFF, TAIL)])


@functools.lru_cache(maxsize=1)
def _sc_scatter_kernel():
    return pl.kernel(
        _sc_scatter_body,
        out_type=jax.ShapeDtypeStruct((2, N, H), jnp.float32),
        mesh=plsc.VectorSubcoreMesh(core_axis_name="c", subcore_axis_name="s",
                                    num_cores=NC, num_subcores=NS),
        scratch_types=[
            pltpu.VMEM((2, CH, EB), jnp.int32),
            pltpu.VMEM((2, CH, EB), jnp.int32),
            tuple(pltpu.VMEM((EB, H), jnp.float32) for _ in range(NBUF)),
            pltpu.VMEM_SHARED((NA, H), jnp.float32),
            tuple(pltpu.SemaphoreType.DMA for _ in range(NBUF)),
            tuple(pltpu.SemaphoreType.DMA for _ in range(NBUF)),
            pltpu.SemaphoreType.DMA,
        ],
    )


def _sc_scatter(h, zeros, src2, dst2):
    return _sc_scatter_kernel()(h, zeros, src2, dst2)


def _mlp_body(p0_ref, p1_ref, W_ref, b_ref, s_ref, be_ref, W2_ref, b2_ref, o_ref):
    a = p0_ref[...] + p1_ref[...]
    t = jnp.dot(a, W_ref[...], preferred_element_type=jnp.float32) + b_ref[...]
    t = _leaky(t)
    t = t * s_ref[...] + be_ref[...]
    t = jnp.dot(t, W2_ref[...], preferred_element_type=jnp.float32) + b2_ref[...]
    o_ref[...] = _leaky(t)


def _mlp(p0, p1, W, b, s, be, W2, b2):
    full = lambda shp: pl.BlockSpec(shp, lambda i: (0,) * len(shp))
    return pl.pallas_call(
        _mlp_body,
        grid=(NBLK,),
        in_specs=[
            pl.BlockSpec((BR, H), lambda i: (i, 0)),
            pl.BlockSpec((BR, H), lambda i: (i, 0)),
            full((H, H)), full((1, H)), full((1, H)), full((1, H)),
            full((H, H)), full((1, H)),
        ],
        out_specs=pl.BlockSpec((BR, H), lambda i: (i, 0)),
        out_shape=jax.ShapeDtypeStruct((N, H), jnp.float32),
    )(p0, p1, W, b, s, be, W2, b2)


def _mlp_pool_body(p0_ref, p1_ref, batch_ref, W_ref, b_ref, s_ref, be_ref,
                   W2_ref, b2_ref, o_ref):
    a = p0_ref[...] + p1_ref[...]
    t = jnp.dot(a, W_ref[...], preferred_element_type=jnp.float32) + b_ref[...]
    t = _leaky(t)
    t = t * s_ref[...] + be_ref[...]
    t = jnp.dot(t, W2_ref[...], preferred_element_type=jnp.float32) + b2_ref[...]
    t = _leaky(t)
    bb = batch_ref[0]
    onehot = (lax.broadcasted_iota(jnp.int32, (NG, BR), 0) == bb)
    contrib = jnp.dot(onehot.astype(jnp.float32), t,
                      preferred_element_type=jnp.float32)

    @pl.when(pl.program_id(0) == 0)
    def _():
        o_ref[...] = contrib

    @pl.when(pl.program_id(0) != 0)
    def _():
        o_ref[...] += contrib


def _mlp_pool(p0, p1, batch3, W, b, s, be, W2, b2):
    full = lambda shp: pl.BlockSpec(shp, lambda i: (0,) * len(shp))
    return pl.pallas_call(
        _mlp_pool_body,
        grid=(NBLK,),
        in_specs=[
            pl.BlockSpec((BR, H), lambda i: (i, 0)),
            pl.BlockSpec((BR, H), lambda i: (i, 0)),
            pl.BlockSpec((1, 1, BR), lambda i: (i, 0, 0)),
            full((H, H)), full((1, H)), full((1, H)), full((1, H)),
            full((H, H)), full((1, H)),
        ],
        out_specs=pl.BlockSpec((NG, H), lambda i: (0, 0)),
        out_shape=jax.ShapeDtypeStruct((NG, H), jnp.float32),
    )(p0, p1, batch3, W, b, s, be, W2, b2)


def _decoder_body(g_ref, cond_ref, es_ref, eb_ref, fcW_ref, fcb_ref,
                  muW_ref, mub_ref, c0W_ref, c0b_ref, c1W_ref, c1b_ref,
                  d0Wz_ref, d0Wc_ref, d0b_ref, s0_ref, b0_ref,
                  d1Wh_ref, d1Wc_ref, d1b_ref, s1_ref, b1_ref,
                  Wd_ref, gdt_ref, P_ref, o_ref):
    f32 = jnp.float32
    dot = lambda a, b: jnp.dot(a, b, preferred_element_type=f32)
    gb = g_ref[...] * es_ref[...] + eb_ref[...]
    gf = dot(gb, fcW_ref[...]) + fcb_ref[...]
    z = dot(gf, muW_ref[...]) + mub_ref[...]
    c = jnp.maximum(dot(cond_ref[...], c0W_ref[...]) + c0b_ref[...], 0.0)
    c = dot(c, c1W_ref[...]) + c1b_ref[...]
    h0 = jnp.maximum(dot(z, d0Wz_ref[...]) + dot(c, d0Wc_ref[...])
                     + d0b_ref[...], 0.0)
    h0 = h0 * s0_ref[...] + b0_ref[...]
    h1 = jnp.maximum(dot(h0, d1Wh_ref[...]) + dot(c, d1Wc_ref[...])
                     + d1b_ref[...], 0.0)
    h1 = h1 * s1_ref[...] + b1_ref[...]
    delta = dot(h1, Wd_ref[...]) + gdt_ref[...]
    vals = (delta >= 0).astype(jnp.bfloat16)
    o_ref[...] = dot(vals, P_ref[...])


def _decoder(g, cond, es, eb, fcW, fcb, muW, mub, c0W, c0b, c1W, c1b,
             d0Wz, d0Wc, d0b, s0, b0, d1Wh, d1Wc, d1b, s1, b1, Wd, gdt, P):
    return pl.pallas_call(
        _decoder_body,
        out_shape=jax.ShapeDtypeStruct((NG, ADJF), jnp.float32),
    )(g, cond, es, eb, fcW, fcb, muW, mub, c0W, c0b, c1W, c1b,
      d0Wz, d0Wc, d0b, s0, b0, d1Wh, d1Wc, d1b, s1, b1, Wd, gdt, P)


def kernel(x, cond, params, edge_index, batch):
    p = params
    f32 = jnp.float32
    row = lambda v: v.reshape(1, -1).astype(f32)

    jpt = KB * EB - EPT
    jdst = N + (jnp.arange(jpt, dtype=jnp.int32) & (NJUNK - 1))
    src2 = jnp.concatenate(
        [edge_index[0].reshape(NW, EPT),
         jnp.zeros((NW, jpt), jnp.int32)], axis=1).reshape(NW * KB, EB)
    dst2 = jnp.concatenate(
        [edge_index[1].reshape(NW, EPT),
         jnp.broadcast_to(jdst, (NW, jpt))], axis=1).reshape(NW * KB, EB)
    zeros = jnp.zeros((N, H), f32)
    batch3 = batch.reshape(NBLK, 1, BR)

    pp = _sc_scatter(x, zeros, src2, dst2)
    h1 = _mlp(pp[0], pp[1], p['c0W'], row(p['c0b']), row(p['c0g'] * _BNS),
              row(p['c0be']), p['c0W2'], row(p['c0b2']))
    pp2 = _sc_scatter(h1, zeros, src2, dst2)
    g = _mlp_pool(pp2[0], pp2[1], batch3, p['c1W'], row(p['c1b']),
                  row(p['c1g'] * _BNS), row(p['c1be']), p['c1W2'], row(p['c1b2']))

    Wd = p['d2W'][:, 0::2] - p['d2W'][:, 1::2]
    Wd = jnp.pad(Wd, ((0, 0), (0, AHP - AH)))
    gn = jax.random.gumbel(jax.random.key(42), (NG, AH, 2), jnp.float32)
    gdiff = gn[:, :, 0] - gn[:, :, 1] + (p['d2b'][0::2] - p['d2b'][1::2])[None, :]
    gdt = jnp.pad(gdiff, ((0, 0), (0, AHP - AH)), constant_values=-1e9)
    P = jnp.asarray(_PFULL, jnp.bfloat16)

    adjf = _decoder(
        g, cond, row(p['ebn_g'] * _BNS), row(p['ebn_b']),
        p['fcW'], row(p['fcb']), p['muW'], row(p['mub']),
        p['cm0W'], row(p['cm0b']), p['cm1W'], row(p['cm1b']),
        p['d0W'][:LAT], p['d0W'][LAT:], row(p['d0b']),
        row(p['dbn0_g'] * _BNS), row(p['dbn0_b']),
        p['d1W'][:HD], p['d1W'][HD:], row(p['d1b']),
        row(p['dbn1_g'] * _BNS), row(p['dbn1_b']),
        Wd, gdt, P)
    return adjf.reshape(NG, NMAX, NMAX)


# speedup vs baseline: 2.8755x; 2.3880x over previous
import functools
import math

import numpy as np
import jax
import jax.numpy as jnp
from jax import lax
from jax.experimental import pallas as pl
from jax.experimental.pallas import tpu as pltpu
from jax.experimental.pallas import tpu_sc as plsc

N = 10000
E = 320000
H = 128
LAT = 32
HD = 256
NMAX = 50
DC = 128
NG = 200
AH = NMAX * (NMAX - 1) // 2
AHP = 1280
ADJF = NMAX * NMAX

NC, NS = 2, 16
NW = NC * NS
EPT = E // NW
EB = 125
KB = EPT // EB
CH = 16
NCH = KB // CH
RPT = 624
TOFF = NS * RPT
TAIL = N - TOFF

BR = 1000
NBLK = N // BR

_BNS = 1.0 / math.sqrt(1.0 + 1e-5)

_IU = np.triu_indices(NMAX, 1)
_PFULL = np.zeros((AHP, ADJF), np.float32)
_PFULL[np.arange(AH), _IU[0] * NMAX + _IU[1]] = 1.0
_PFULL[np.arange(AH), _IU[1] * NMAX + _IU[0]] = 1.0


def _leaky(t):
    return jnp.where(t > 0, t, 0.2 * t)


def _sc_scatter_body(h_hbm, z_hbm, src_hbm, dst_hbm, out_hbm,
                     src_v, dst_v, bufs, agg, semsG, semsS):
    buf = list(bufs)
    semG = list(semsG)
    semS = list(semsS)
    cid = lax.axis_index("c")
    sid = lax.axis_index("s")
    wid = cid * NS + sid
    row0 = sid * RPT

    @pl.when(cid == 0)
    def _():
        pltpu.sync_copy(h_hbm.at[pl.ds(row0, RPT)], agg.at[pl.ds(row0, RPT)])

        @pl.when(sid == NS - 1)
        def _():
            pltpu.sync_copy(h_hbm.at[pl.ds(TOFF, TAIL)], agg.at[pl.ds(TOFF, TAIL)])

    @pl.when(cid != 0)
    def _():
        pltpu.sync_copy(z_hbm.at[pl.ds(row0, RPT)], agg.at[pl.ds(row0, RPT)])

        @pl.when(sid == NS - 1)
        def _():
            pltpu.sync_copy(z_hbm.at[pl.ds(TOFF, TAIL)], agg.at[pl.ds(TOFF, TAIL)])

    plsc.subcore_barrier()

    @pl.loop(0, NCH)
    def _(ci):
        cbase = wid * KB + ci * CH
        pltpu.sync_copy(src_hbm.at[pl.ds(cbase, CH)], src_v)
        pltpu.sync_copy(dst_hbm.at[pl.ds(cbase, CH)], dst_v)
        pltpu.async_copy(h_hbm.at[src_v.at[0]], buf[0], semG[0])

        for b in range(CH):
            j = b % 2
            pltpu.make_async_copy(h_hbm.at[src_v.at[b]], buf[j], semG[j]).wait()
            if b >= 1:
                pltpu.make_async_copy(
                    buf[1 - j], agg.at[dst_v.at[b - 1]], semS[1 - j]).wait()
            pltpu.async_copy(buf[j], agg.at[dst_v.at[b]], semS[j], add=True)
            if b + 1 < CH:
                pltpu.async_copy(h_hbm.at[src_v.at[b + 1]], buf[1 - j], semG[1 - j])

        pltpu.make_async_copy(
            buf[(CH - 1) % 2], agg.at[dst_v.at[CH - 1]], semS[(CH - 1) % 2]).wait()

    plsc.subcore_barrier()

    pltpu.sync_copy(agg.at[pl.ds(row0, RPT)], out_hbm.at[cid, pl.ds(row0, RPT)])

    @pl.when(sid == NS - 1)
    def _():
        pltpu.sync_copy(agg.at[pl.ds(TOFF, TAIL)], out_hbm.at[cid, pl.ds(TOFF, TAIL)])


@functools.lru_cache(maxsize=1)
def _sc_scatter_kernel():
    return pl.kernel(
        _sc_scatter_body,
        out_type=jax.ShapeDtypeStruct((2, N, H), jnp.float32),
        mesh=plsc.VectorSubcoreMesh(core_axis_name="c", subcore_axis_name="s",
                                    num_cores=NC, num_subcores=NS),
        scratch_types=[
            pltpu.VMEM((CH, EB), jnp.int32),
            pltpu.VMEM((CH, EB), jnp.int32),
            tuple(pltpu.VMEM((EB, H), jnp.float32) for _ in range(2)),
            pltpu.VMEM_SHARED((N, H), jnp.float32),
            tuple(pltpu.SemaphoreType.DMA for _ in range(2)),
            tuple(pltpu.SemaphoreType.DMA for _ in range(2)),
        ],
    )


def _sc_scatter(h, zeros, src2, dst2):
    return _sc_scatter_kernel()(h, zeros, src2, dst2)


def _mlp_body(p0_ref, p1_ref, W_ref, b_ref, s_ref, be_ref, W2_ref, b2_ref, o_ref):
    a = p0_ref[...] + p1_ref[...]
    t = jnp.dot(a, W_ref[...], preferred_element_type=jnp.float32) + b_ref[...]
    t = _leaky(t)
    t = t * s_ref[...] + be_ref[...]
    t = jnp.dot(t, W2_ref[...], preferred_element_type=jnp.float32) + b2_ref[...]
    o_ref[...] = _leaky(t)


def _mlp(p0, p1, W, b, s, be, W2, b2):
    full = lambda shp: pl.BlockSpec(shp, lambda i: (0,) * len(shp))
    return pl.pallas_call(
        _mlp_body,
        grid=(NBLK,),
        in_specs=[
            pl.BlockSpec((BR, H), lambda i: (i, 0)),
            pl.BlockSpec((BR, H), lambda i: (i, 0)),
            full((H, H)), full((1, H)), full((1, H)), full((1, H)),
            full((H, H)), full((1, H)),
        ],
        out_specs=pl.BlockSpec((BR, H), lambda i: (i, 0)),
        out_shape=jax.ShapeDtypeStruct((N, H), jnp.float32),
    )(p0, p1, W, b, s, be, W2, b2)


def _mlp_pool_body(p0_ref, p1_ref, batch_ref, W_ref, b_ref, s_ref, be_ref,
                   W2_ref, b2_ref, o_ref):
    a = p0_ref[...] + p1_ref[...]
    t = jnp.dot(a, W_ref[...], preferred_element_type=jnp.float32) + b_ref[...]
    t = _leaky(t)
    t = t * s_ref[...] + be_ref[...]
    t = jnp.dot(t, W2_ref[...], preferred_element_type=jnp.float32) + b2_ref[...]
    t = _leaky(t)
    bb = batch_ref[0]
    onehot = (lax.broadcasted_iota(jnp.int32, (NG, BR), 0) == bb)
    contrib = jnp.dot(onehot.astype(jnp.float32), t,
                      preferred_element_type=jnp.float32)

    @pl.when(pl.program_id(0) == 0)
    def _():
        o_ref[...] = contrib

    @pl.when(pl.program_id(0) != 0)
    def _():
        o_ref[...] += contrib


def _mlp_pool(p0, p1, batch3, W, b, s, be, W2, b2):
    full = lambda shp: pl.BlockSpec(shp, lambda i: (0,) * len(shp))
    return pl.pallas_call(
        _mlp_pool_body,
        grid=(NBLK,),
        in_specs=[
            pl.BlockSpec((BR, H), lambda i: (i, 0)),
            pl.BlockSpec((BR, H), lambda i: (i, 0)),
            pl.BlockSpec((1, 1, BR), lambda i: (i, 0, 0)),
            full((H, H)), full((1, H)), full((1, H)), full((1, H)),
            full((H, H)), full((1, H)),
        ],
        out_specs=pl.BlockSpec((NG, H), lambda i: (0, 0)),
        out_shape=jax.ShapeDtypeStruct((NG, H), jnp.float32),
    )(p0, p1, batch3, W, b, s, be, W2, b2)


def _decoder_body(g_ref, cond_ref, es_ref, eb_ref, fcW_ref, fcb_ref,
                  muW_ref, mub_ref, c0W_ref, c0b_ref, c1W_ref, c1b_ref,
                  d0Wz_ref, d0Wc_ref, d0b_ref, s0_ref, b0_ref,
                  d1Wh_ref, d1Wc_ref, d1b_ref, s1_ref, b1_ref,
                  Wd_ref, gdt_ref, P_ref, o_ref):
    f32 = jnp.float32
    dot = lambda a, b: jnp.dot(a, b, preferred_element_type=f32)
    gb = g_ref[...] * es_ref[...] + eb_ref[...]
    gf = dot(gb, fcW_ref[...]) + fcb_ref[...]
    z = dot(gf, muW_ref[...]) + mub_ref[...]
    c = jnp.maximum(dot(cond_ref[...], c0W_ref[...]) + c0b_ref[...], 0.0)
    c = dot(c, c1W_ref[...]) + c1b_ref[...]
    h0 = jnp.maximum(dot(z, d0Wz_ref[...]) + dot(c, d0Wc_ref[...])
                     + d0b_ref[...], 0.0)
    h0 = h0 * s0_ref[...] + b0_ref[...]
    h1 = jnp.maximum(dot(h0, d1Wh_ref[...]) + dot(c, d1Wc_ref[...])
                     + d1b_ref[...], 0.0)
    h1 = h1 * s1_ref[...] + b1_ref[...]
    delta = dot(h1, Wd_ref[...]) + gdt_ref[...]
    vals = (delta >= 0).astype(jnp.bfloat16)
    o_ref[...] = dot(vals, P_ref[...])


def _decoder(g, cond, es, eb, fcW, fcb, muW, mub, c0W, c0b, c1W, c1b,
             d0Wz, d0Wc, d0b, s0, b0, d1Wh, d1Wc, d1b, s1, b1, Wd, gdt, P):
    return pl.pallas_call(
        _decoder_body,
        out_shape=jax.ShapeDtypeStruct((NG, ADJF), jnp.float32),
    )(g, cond, es, eb, fcW, fcb, muW, mub, c0W, c0b, c1W, c1b,
      d0Wz, d0Wc, d0b, s0, b0, d1Wh, d1Wc, d1b, s1, b1, Wd, gdt, P)


def kernel(x, cond, params, edge_index, batch):
    p = params
    f32 = jnp.float32
    row = lambda v: v.reshape(1, -1).astype(f32)

    src2 = edge_index[0].reshape(NW * KB, EB)
    dst2 = edge_index[1].reshape(NW * KB, EB)
    zeros = jnp.zeros((N, H), f32)
    batch3 = batch.reshape(NBLK, 1, BR)

    pp = _sc_scatter(x, zeros, src2, dst2)
    h1 = _mlp(pp[0], pp[1], p['c0W'], row(p['c0b']), row(p['c0g'] * _BNS),
              row(p['c0be']), p['c0W2'], row(p['c0b2']))
    pp2 = _sc_scatter(h1, zeros, src2, dst2)
    g = _mlp_pool(pp2[0], pp2[1], batch3, p['c1W'], row(p['c1b']),
                  row(p['c1g'] * _BNS), row(p['c1be']), p['c1W2'], row(p['c1b2']))

    Wd = p['d2W'][:, 0::2] - p['d2W'][:, 1::2]
    Wd = jnp.pad(Wd, ((0, 0), (0, AHP - AH)))
    gn = jax.random.gumbel(jax.random.key(42), (NG, AH, 2), jnp.float32)
    gdiff = gn[:, :, 0] - gn[:, :, 1] + (p['d2b'][0::2] - p['d2b'][1::2])[None, :]
    gdt = jnp.pad(gdiff, ((0, 0), (0, AHP - AH)), constant_values=-1e9)
    P = jnp.asarray(_PFULL, jnp.bfloat16)

    adjf = _decoder(
        g, cond, row(p['ebn_g'] * _BNS), row(p['ebn_b']),
        p['fcW'], row(p['fcb']), p['muW'], row(p['mub']),
        p['cm0W'], row(p['cm0b']), p['cm1W'], row(p['cm1b']),
        p['d0W'][:LAT], p['d0W'][LAT:], row(p['d0b']),
        row(p['dbn0_g'] * _BNS), row(p['dbn0_b']),
        p['d1W'][:HD], p['d1W'][HD:], row(p['d1b']),
        row(p['dbn1_g'] * _BNS), row(p['dbn1_b']),
        Wd, gdt, P)
    return adjf.reshape(NG, NMAX, NMAX)


# speedup vs baseline: 3.1991x; 1.1125x over previous
import functools
import math

import numpy as np
import jax
import jax.numpy as jnp
from jax import lax
from jax.experimental import pallas as pl
from jax.experimental.pallas import tpu as pltpu
from jax.experimental.pallas import tpu_sc as plsc

N = 10000
E = 320000
H = 128
LAT = 32
HD = 256
NMAX = 50
DC = 128
NG = 200
AH = NMAX * (NMAX - 1) // 2
AHP = 1280
ADJF = NMAX * NMAX

NC, NS = 2, 16
NW = NC * NS
EPT = E // NW
EB = 125
KB = EPT // EB
CH = 16
NCH = KB // CH
RPT = 624
TOFF = NS * RPT
TAIL = N - TOFF

BR = 1000
NBLK = N // BR

_BNS = 1.0 / math.sqrt(1.0 + 1e-5)

def _threefry2x32_np(k0, k1, x0, x1):
    rot = [[13, 15, 26, 6], [17, 29, 16, 24]]

    def rotl(v, d):
        return ((v << np.uint32(d)) | (v >> np.uint32(32 - d))).astype(np.uint32)

    ks = [k0, k1, np.uint32(0x1BD11BDA) ^ k0 ^ k1]
    x0 = (x0 + ks[0]).astype(np.uint32)
    x1 = (x1 + ks[1]).astype(np.uint32)
    for r in range(5):
        for d in rot[r % 2]:
            x0 = (x0 + x1).astype(np.uint32)
            x1 = rotl(x1, d) ^ x0
        x0 = (x0 + ks[(r + 1) % 3]).astype(np.uint32)
        x1 = (x1 + ks[(r + 2) % 3] + np.uint32(r + 1)).astype(np.uint32)
    return x0, x1


def _gumbel_np(seed, shape):
    n = int(np.prod(shape))
    b0, b1 = _threefry2x32_np(np.uint32(seed >> 32), np.uint32(seed & 0xFFFFFFFF),
                              np.zeros(n, np.uint32), np.arange(n, dtype=np.uint32))
    bits = b0 ^ b1
    u = ((bits >> np.uint32(9)) | np.uint32(0x3F800000)).view(np.float32) \
        - np.float32(1.0)
    tiny = np.float32(np.finfo(np.float32).tiny)
    u = np.maximum(tiny, (u * (np.float32(1.0) - tiny) + tiny).astype(np.float32))
    return (-np.log(-np.log(u))).astype(np.float32).reshape(shape)


_GN = _gumbel_np(42, (NG, AH, 2))
_GDIFF = _GN[:, :, 0] - _GN[:, :, 1]

_IU = np.triu_indices(NMAX, 1)
_PFULL = np.zeros((AHP, ADJF), np.float32)
_PFULL[np.arange(AH), _IU[0] * NMAX + _IU[1]] = 1.0
_PFULL[np.arange(AH), _IU[1] * NMAX + _IU[0]] = 1.0


def _leaky(t):
    return jnp.where(t > 0, t, 0.2 * t)


def _sc_scatter_body(h_hbm, z_hbm, src_hbm, dst_hbm, out_hbm,
                     src_v, dst_v, bufs, agg, semsG, semsS):
    buf = list(bufs)
    semG = list(semsG)
    semS = list(semsS)
    cid = lax.axis_index("c")
    sid = lax.axis_index("s")
    wid = cid * NS + sid
    row0 = sid * RPT

    @pl.when(cid == 0)
    def _():
        pltpu.sync_copy(h_hbm.at[pl.ds(row0, RPT)], agg.at[pl.ds(row0, RPT)])

        @pl.when(sid == NS - 1)
        def _():
            pltpu.sync_copy(h_hbm.at[pl.ds(TOFF, TAIL)], agg.at[pl.ds(TOFF, TAIL)])

    @pl.when(cid != 0)
    def _():
        pltpu.sync_copy(z_hbm.at[pl.ds(row0, RPT)], agg.at[pl.ds(row0, RPT)])

        @pl.when(sid == NS - 1)
        def _():
            pltpu.sync_copy(z_hbm.at[pl.ds(TOFF, TAIL)], agg.at[pl.ds(TOFF, TAIL)])

    plsc.subcore_barrier()

    @pl.loop(0, NCH)
    def _(ci):
        cbase = wid * KB + ci * CH
        pltpu.sync_copy(src_hbm.at[pl.ds(cbase, CH)], src_v)
        pltpu.sync_copy(dst_hbm.at[pl.ds(cbase, CH)], dst_v)
        pltpu.async_copy(h_hbm.at[src_v.at[0]], buf[0], semG[0])

        pltpu.async_copy(h_hbm.at[src_v.at[1]], buf[1], semG[1])
        for b in range(CH):
            j = b % 2
            pltpu.make_async_copy(h_hbm.at[src_v.at[b]], buf[j], semG[j]).wait()
            pltpu.sync_copy(buf[j], agg.at[dst_v.at[b]], add=True)
            if b + 2 < CH:
                pltpu.async_copy(h_hbm.at[src_v.at[b + 2]], buf[j], semG[j])

    plsc.subcore_barrier()

    pltpu.sync_copy(agg.at[pl.ds(row0, RPT)], out_hbm.at[cid, pl.ds(row0, RPT)])

    @pl.when(sid == NS - 1)
    def _():
        pltpu.sync_copy(agg.at[pl.ds(TOFF, TAIL)], out_hbm.at[cid, pl.ds(TOFF, TAIL)])


@functools.lru_cache(maxsize=1)
def _sc_scatter_kernel():
    return pl.kernel(
        _sc_scatter_body,
        out_type=jax.ShapeDtypeStruct((2, N, H), jnp.float32),
        mesh=plsc.VectorSubcoreMesh(core_axis_name="c", subcore_axis_name="s",
                                    num_cores=NC, num_subcores=NS),
        scratch_types=[
            pltpu.VMEM((CH, EB), jnp.int32),
            pltpu.VMEM((CH, EB), jnp.int32),
            tuple(pltpu.VMEM((EB, H), jnp.float32) for _ in range(2)),
            pltpu.VMEM_SHARED((N, H), jnp.float32),
            tuple(pltpu.SemaphoreType.DMA for _ in range(2)),
            tuple(pltpu.SemaphoreType.DMA for _ in range(2)),
        ],
    )


def _sc_scatter(h, zeros, src2, dst2):
    return _sc_scatter_kernel()(h, zeros, src2, dst2)


def _mlp_body(p0_ref, p1_ref, W_ref, b_ref, s_ref, be_ref, W2_ref, b2_ref, o_ref):
    a = p0_ref[...] + p1_ref[...]
    t = jnp.dot(a, W_ref[...], preferred_element_type=jnp.float32) + b_ref[...]
    t = _leaky(t)
    t = t * s_ref[...] + be_ref[...]
    t = jnp.dot(t, W2_ref[...], preferred_element_type=jnp.float32) + b2_ref[...]
    o_ref[...] = _leaky(t)


def _mlp(p0, p1, W, b, s, be, W2, b2):
    full = lambda shp: pl.BlockSpec(shp, lambda i: (0,) * len(shp))
    return pl.pallas_call(
        _mlp_body,
        grid=(NBLK,),
        in_specs=[
            pl.BlockSpec((BR, H), lambda i: (i, 0)),
            pl.BlockSpec((BR, H), lambda i: (i, 0)),
            full((H, H)), full((1, H)), full((1, H)), full((1, H)),
            full((H, H)), full((1, H)),
        ],
        out_specs=pl.BlockSpec((BR, H), lambda i: (i, 0)),
        out_shape=jax.ShapeDtypeStruct((N, H), jnp.float32),
    )(p0, p1, W, b, s, be, W2, b2)


def _mlp_pool_body(p0_ref, p1_ref, batch_ref, W_ref, b_ref, s_ref, be_ref,
                   W2_ref, b2_ref, o_ref):
    a = p0_ref[...] + p1_ref[...]
    t = jnp.dot(a, W_ref[...], preferred_element_type=jnp.float32) + b_ref[...]
    t = _leaky(t)
    t = t * s_ref[...] + be_ref[...]
    t = jnp.dot(t, W2_ref[...], preferred_element_type=jnp.float32) + b2_ref[...]
    t = _leaky(t)
    bb = batch_ref[0]
    onehot = (lax.broadcasted_iota(jnp.int32, (NG, BR), 0) == bb)
    contrib = jnp.dot(onehot.astype(jnp.float32), t,
                      preferred_element_type=jnp.float32)

    @pl.when(pl.program_id(0) == 0)
    def _():
        o_ref[...] = contrib

    @pl.when(pl.program_id(0) != 0)
    def _():
        o_ref[...] += contrib


def _mlp_pool(p0, p1, batch3, W, b, s, be, W2, b2):
    full = lambda shp: pl.BlockSpec(shp, lambda i: (0,) * len(shp))
    return pl.pallas_call(
        _mlp_pool_body,
        grid=(NBLK,),
        in_specs=[
            pl.BlockSpec((BR, H), lambda i: (i, 0)),
            pl.BlockSpec((BR, H), lambda i: (i, 0)),
            pl.BlockSpec((1, 1, BR), lambda i: (i, 0, 0)),
            full((H, H)), full((1, H)), full((1, H)), full((1, H)),
            full((H, H)), full((1, H)),
        ],
        out_specs=pl.BlockSpec((NG, H), lambda i: (0, 0)),
        out_shape=jax.ShapeDtypeStruct((NG, H), jnp.float32),
    )(p0, p1, batch3, W, b, s, be, W2, b2)


def _decoder_body(g_ref, cond_ref, es_ref, eb_ref, fcW_ref, fcb_ref,
                  muW_ref, mub_ref, c0W_ref, c0b_ref, c1W_ref, c1b_ref,
                  d0Wz_ref, d0Wc_ref, d0b_ref, s0_ref, b0_ref,
                  d1Wh_ref, d1Wc_ref, d1b_ref, s1_ref, b1_ref,
                  Wd_ref, gdt_ref, P_ref, o_ref):
    f32 = jnp.float32
    dot = lambda a, b: jnp.dot(a, b, preferred_element_type=f32)
    gb = g_ref[...] * es_ref[...] + eb_ref[...]
    gf = dot(gb, fcW_ref[...]) + fcb_ref[...]
    z = dot(gf, muW_ref[...]) + mub_ref[...]
    c = jnp.maximum(dot(cond_ref[...], c0W_ref[...]) + c0b_ref[...], 0.0)
    c = dot(c, c1W_ref[...]) + c1b_ref[...]
    h0 = jnp.maximum(dot(z, d0Wz_ref[...]) + dot(c, d0Wc_ref[...])
                     + d0b_ref[...], 0.0)
    h0 = h0 * s0_ref[...] + b0_ref[...]
    h1 = jnp.maximum(dot(h0, d1Wh_ref[...]) + dot(c, d1Wc_ref[...])
                     + d1b_ref[...], 0.0)
    h1 = h1 * s1_ref[...] + b1_ref[...]
    delta = dot(h1, Wd_ref[...]) + gdt_ref[...]
    vals = (delta >= 0).astype(jnp.bfloat16)
    o_ref[...] = dot(vals, P_ref[...])


def _decoder(g, cond, es, eb, fcW, fcb, muW, mub, c0W, c0b, c1W, c1b,
             d0Wz, d0Wc, d0b, s0, b0, d1Wh, d1Wc, d1b, s1, b1, Wd, gdt, P):
    return pl.pallas_call(
        _decoder_body,
        out_shape=jax.ShapeDtypeStruct((NG, ADJF), jnp.float32),
    )(g, cond, es, eb, fcW, fcb, muW, mub, c0W, c0b, c1W, c1b,
      d0Wz, d0Wc, d0b, s0, b0, d1Wh, d1Wc, d1b, s1, b1, Wd, gdt, P)


def kernel(x, cond, params, edge_index, batch):
    p = params
    f32 = jnp.float32
    row = lambda v: v.reshape(1, -1).astype(f32)

    src2 = edge_index[0].reshape(NW * KB, EB)
    dst2 = edge_index[1].reshape(NW * KB, EB)
    zeros = jnp.zeros((N, H), f32)
    batch3 = batch.reshape(NBLK, 1, BR)

    pp = _sc_scatter(x, zeros, src2, dst2)
    h1 = _mlp(pp[0], pp[1], p['c0W'], row(p['c0b']), row(p['c0g'] * _BNS),
              row(p['c0be']), p['c0W2'], row(p['c0b2']))
    pp2 = _sc_scatter(h1, zeros, src2, dst2)
    g = _mlp_pool(pp2[0], pp2[1], batch3, p['c1W'], row(p['c1b']),
                  row(p['c1g'] * _BNS), row(p['c1be']), p['c1W2'], row(p['c1b2']))

    Wd = p['d2W'][:, 0::2] - p['d2W'][:, 1::2]
    Wd = jnp.pad(Wd, ((0, 0), (0, AHP - AH)))
    gdiff = jnp.asarray(_GDIFF) + (p['d2b'][0::2] - p['d2b'][1::2])[None, :]
    gdt = jnp.pad(gdiff, ((0, 0), (0, AHP - AH)), constant_values=-1e9)
    P = jnp.asarray(_PFULL, jnp.bfloat16)

    adjf = _decoder(
        g, cond, row(p['ebn_g'] * _BNS), row(p['ebn_b']),
        p['fcW'], row(p['fcb']), p['muW'], row(p['mub']),
        p['cm0W'], row(p['cm0b']), p['cm1W'], row(p['cm1b']),
        p['d0W'][:LAT], p['d0W'][LAT:], row(p['d0b']),
        row(p['dbn0_g'] * _BNS), row(p['dbn0_b']),
        p['d1W'][:HD], p['d1W'][HD:], row(p['d1b']),
        row(p['dbn1_g'] * _BNS), row(p['dbn1_b']),
        Wd, gdt, P)
    return adjf.reshape(NG, NMAX, NMAX)


# speedup vs baseline: 3.2133x; 1.0044x over previous
import functools
import math

import numpy as np
import jax
import jax.numpy as jnp
from jax import lax
from jax.experimental import pallas as pl
from jax.experimental.pallas import tpu as pltpu
from jax.experimental.pallas import tpu_sc as plsc

N = 10000
E = 320000
H = 128
LAT = 32
HD = 256
NMAX = 50
DC = 128
NG = 200
AH = NMAX * (NMAX - 1) // 2
AHP = 1280
ADJF = NMAX * NMAX

NC, NS = 2, 16
NW = NC * NS
EPT = E // NW
EB = 125
KB = EPT // EB
CH = 16
NCH = KB // CH
RPT = 624
TOFF = NS * RPT
TAIL = N - TOFF

BR = 1000
NBLK = N // BR

_BNS = 1.0 / math.sqrt(1.0 + 1e-5)

def _threefry2x32_np(k0, k1, x0, x1):
    rot = [[13, 15, 26, 6], [17, 29, 16, 24]]

    def rotl(v, d):
        return ((v << np.uint32(d)) | (v >> np.uint32(32 - d))).astype(np.uint32)

    ks = [k0, k1, np.uint32(0x1BD11BDA) ^ k0 ^ k1]
    x0 = (x0 + ks[0]).astype(np.uint32)
    x1 = (x1 + ks[1]).astype(np.uint32)
    for r in range(5):
        for d in rot[r % 2]:
            x0 = (x0 + x1).astype(np.uint32)
            x1 = rotl(x1, d) ^ x0
        x0 = (x0 + ks[(r + 1) % 3]).astype(np.uint32)
        x1 = (x1 + ks[(r + 2) % 3] + np.uint32(r + 1)).astype(np.uint32)
    return x0, x1


def _gumbel_np(seed, shape):
    n = int(np.prod(shape))
    b0, b1 = _threefry2x32_np(np.uint32(seed >> 32), np.uint32(seed & 0xFFFFFFFF),
                              np.zeros(n, np.uint32), np.arange(n, dtype=np.uint32))
    bits = b0 ^ b1
    u = ((bits >> np.uint32(9)) | np.uint32(0x3F800000)).view(np.float32) \
        - np.float32(1.0)
    tiny = np.float32(np.finfo(np.float32).tiny)
    u = np.maximum(tiny, (u * (np.float32(1.0) - tiny) + tiny).astype(np.float32))
    return (-np.log(-np.log(u))).astype(np.float32).reshape(shape)


_GN = _gumbel_np(42, (NG, AH, 2))
_GDIFF = _GN[:, :, 0] - _GN[:, :, 1]

_IU = np.triu_indices(NMAX, 1)
_PFULL = np.zeros((AHP, ADJF), np.float32)
_PFULL[np.arange(AH), _IU[0] * NMAX + _IU[1]] = 1.0
_PFULL[np.arange(AH), _IU[1] * NMAX + _IU[0]] = 1.0


def _leaky(t):
    return jnp.where(t > 0, t, 0.2 * t)


def _sc_scatter_body(h_hbm, z_hbm, src_hbm, dst_hbm, out_hbm,
                     src_v, dst_v, bufs, agg, semsG, semsS):
    buf = list(bufs)
    semG = list(semsG)
    semS = list(semsS)
    cid = lax.axis_index("c")
    sid = lax.axis_index("s")
    wid = cid * NS + sid
    row0 = sid * RPT

    @pl.when(cid == 0)
    def _():
        pltpu.sync_copy(h_hbm.at[pl.ds(row0, RPT)], agg.at[pl.ds(row0, RPT)])

        @pl.when(sid == NS - 1)
        def _():
            pltpu.sync_copy(h_hbm.at[pl.ds(TOFF, TAIL)], agg.at[pl.ds(TOFF, TAIL)])

    @pl.when(cid != 0)
    def _():
        pltpu.sync_copy(z_hbm.at[pl.ds(row0, RPT)], agg.at[pl.ds(row0, RPT)])

        @pl.when(sid == NS - 1)
        def _():
            pltpu.sync_copy(z_hbm.at[pl.ds(TOFF, TAIL)], agg.at[pl.ds(TOFF, TAIL)])

    plsc.subcore_barrier()

    @pl.loop(0, NCH)
    def _(ci):
        cbase = wid * KB + ci * CH
        pltpu.sync_copy(src_hbm.at[pl.ds(cbase, CH)], src_v)
        pltpu.sync_copy(dst_hbm.at[pl.ds(cbase, CH)], dst_v)
        pltpu.async_copy(h_hbm.at[src_v.at[0]], buf[0], semG[0])

        pltpu.async_copy(h_hbm.at[src_v.at[1]], buf[1], semG[1])
        for b in range(CH):
            j = b % 2
            pltpu.make_async_copy(h_hbm.at[src_v.at[b]], buf[j], semG[j]).wait()
            pltpu.sync_copy(buf[j], agg.at[dst_v.at[b]], add=True)
            if b + 2 < CH:
                pltpu.async_copy(h_hbm.at[src_v.at[b + 2]], buf[j], semG[j])

    plsc.subcore_barrier()

    pltpu.sync_copy(agg.at[pl.ds(row0, RPT)], out_hbm.at[cid, pl.ds(row0, RPT)])

    @pl.when(sid == NS - 1)
    def _():
        pltpu.sync_copy(agg.at[pl.ds(TOFF, TAIL)], out_hbm.at[cid, pl.ds(TOFF, TAIL)])


@functools.lru_cache(maxsize=1)
def _sc_scatter_kernel():
    return pl.kernel(
        _sc_scatter_body,
        out_type=jax.ShapeDtypeStruct((2, N, H), jnp.float32),
        mesh=plsc.VectorSubcoreMesh(core_axis_name="c", subcore_axis_name="s",
                                    num_cores=NC, num_subcores=NS),
        scratch_types=[
            pltpu.VMEM((CH, EB), jnp.int32),
            pltpu.VMEM((CH, EB), jnp.int32),
            tuple(pltpu.VMEM((EB, H), jnp.float32) for _ in range(2)),
            pltpu.VMEM_SHARED((N, H), jnp.float32),
            tuple(pltpu.SemaphoreType.DMA for _ in range(2)),
            tuple(pltpu.SemaphoreType.DMA for _ in range(2)),
        ],
    )


def _sc_scatter(h, zeros, src2, dst2):
    return _sc_scatter_kernel()(h, zeros, src2, dst2)


def _mlp_body(p0_ref, p1_ref, W_ref, b_ref, s_ref, be_ref, W2_ref, b2_ref, o_ref):
    a = p0_ref[...] + p1_ref[...]
    t = jnp.dot(a, W_ref[...], preferred_element_type=jnp.float32) + b_ref[...]
    t = _leaky(t)
    t = t * s_ref[...] + be_ref[...]
    t = jnp.dot(t, W2_ref[...], preferred_element_type=jnp.float32) + b2_ref[...]
    o_ref[...] = _leaky(t)


def _mlp(p0, p1, W, b, s, be, W2, b2):
    full = lambda shp: pl.BlockSpec(shp, lambda i: (0,) * len(shp))
    return pl.pallas_call(
        _mlp_body,
        grid=(NBLK,),
        in_specs=[
            pl.BlockSpec((BR, H), lambda i: (i, 0)),
            pl.BlockSpec((BR, H), lambda i: (i, 0)),
            full((H, H)), full((1, H)), full((1, H)), full((1, H)),
            full((H, H)), full((1, H)),
        ],
        out_specs=pl.BlockSpec((BR, H), lambda i: (i, 0)),
        out_shape=jax.ShapeDtypeStruct((N, H), jnp.float32),
    )(p0, p1, W, b, s, be, W2, b2)


def _mlp_pool_dec_body(p0_ref, p1_ref, batch_ref, W_ref, b_ref, s_ref, be_ref,
                       W2_ref, b2_ref, cond_ref, es_ref, eb_ref, fcW_ref,
                       fcb_ref, muW_ref, mub_ref, c0W_ref, c0b_ref, c1W_ref,
                       c1b_ref, d0Wz_ref, d0Wc_ref, d0b_ref, s0_ref, b0_ref,
                       d1Wh_ref, d1Wc_ref, d1b_ref, s1_ref, b1_ref,
                       Wd_ref, gdt_ref, P_ref, o_ref, g_acc):
    f32 = jnp.float32
    dot = lambda a, b: jnp.dot(a, b, preferred_element_type=f32)
    a = p0_ref[...] + p1_ref[...]
    t = dot(a, W_ref[...]) + b_ref[...]
    t = _leaky(t)
    t = t * s_ref[...] + be_ref[...]
    t = dot(t, W2_ref[...]) + b2_ref[...]
    t = _leaky(t)
    bb = batch_ref[0]
    onehot = (lax.broadcasted_iota(jnp.int32, (NG, BR), 0) == bb)
    contrib = dot(onehot.astype(f32), t)

    @pl.when(pl.program_id(0) == 0)
    def _():
        g_acc[...] = contrib

    @pl.when(pl.program_id(0) != 0)
    def _():
        g_acc[...] += contrib

    @pl.when(pl.program_id(0) == NBLK - 1)
    def _():
        gb = g_acc[...] * es_ref[...] + eb_ref[...]
        gf = dot(gb, fcW_ref[...]) + fcb_ref[...]
        z = dot(gf, muW_ref[...]) + mub_ref[...]
        c = jnp.maximum(dot(cond_ref[...], c0W_ref[...]) + c0b_ref[...], 0.0)
        c = dot(c, c1W_ref[...]) + c1b_ref[...]
        h0 = jnp.maximum(dot(z, d0Wz_ref[...]) + dot(c, d0Wc_ref[...])
                         + d0b_ref[...], 0.0)
        h0 = h0 * s0_ref[...] + b0_ref[...]
        h1 = jnp.maximum(dot(h0, d1Wh_ref[...]) + dot(c, d1Wc_ref[...])
                         + d1b_ref[...], 0.0)
        h1 = h1 * s1_ref[...] + b1_ref[...]
        delta = dot(h1, Wd_ref[...]) + gdt_ref[...]
        vals = (delta >= 0).astype(jnp.bfloat16)
        o_ref[...] = dot(vals, P_ref[...])


def _mlp_pool_dec(p0, p1, batch3, W, b, s, be, W2, b2, *dec):
    full = lambda shp: pl.BlockSpec(shp, lambda i: (0,) * len(shp))
    return pl.pallas_call(
        _mlp_pool_dec_body,
        grid=(NBLK,),
        in_specs=[
            pl.BlockSpec((BR, H), lambda i: (i, 0)),
            pl.BlockSpec((BR, H), lambda i: (i, 0)),
            pl.BlockSpec((1, 1, BR), lambda i: (i, 0, 0)),
            full((H, H)), full((1, H)), full((1, H)), full((1, H)),
            full((H, H)), full((1, H)),
        ] + [full(d.shape) for d in dec],
        out_specs=pl.BlockSpec((NG, ADJF), lambda i: (0, 0)),
        out_shape=jax.ShapeDtypeStruct((NG, ADJF), jnp.float32),
        scratch_shapes=[pltpu.VMEM((NG, H), jnp.float32)],
    )(p0, p1, batch3, W, b, s, be, W2, b2, *dec)


def kernel(x, cond, params, edge_index, batch):
    p = params
    f32 = jnp.float32
    row = lambda v: v.reshape(1, -1).astype(f32)

    src2 = edge_index[0].reshape(NW * KB, EB)
    dst2 = edge_index[1].reshape(NW * KB, EB)
    zeros = jnp.zeros((N, H), f32)
    batch3 = batch.reshape(NBLK, 1, BR)

    pp = _sc_scatter(x, zeros, src2, dst2)
    h1 = _mlp(pp[0], pp[1], p['c0W'], row(p['c0b']), row(p['c0g'] * _BNS),
              row(p['c0be']), p['c0W2'], row(p['c0b2']))
    Wd = p['d2W'][:, 0::2] - p['d2W'][:, 1::2]
    Wd = jnp.pad(Wd, ((0, 0), (0, AHP - AH)))
    gdiff = jnp.asarray(_GDIFF) + (p['d2b'][0::2] - p['d2b'][1::2])[None, :]
    gdt = jnp.pad(gdiff, ((0, 0), (0, AHP - AH)), constant_values=-1e9)
    P = jnp.asarray(_PFULL, jnp.bfloat16)

    pp2 = _sc_scatter(h1, zeros, src2, dst2)
    adjf = _mlp_pool_dec(
        pp2[0], pp2[1], batch3, p['c1W'], row(p['c1b']),
        row(p['c1g'] * _BNS), row(p['c1be']), p['c1W2'], row(p['c1b2']),
        cond, row(p['ebn_g'] * _BNS), row(p['ebn_b']),
        p['fcW'], row(p['fcb']), p['muW'], row(p['mub']),
        p['cm0W'], row(p['cm0b']), p['cm1W'], row(p['cm1b']),
        p['d0W'][:LAT], p['d0W'][LAT:], row(p['d0b']),
        row(p['dbn0_g'] * _BNS), row(p['dbn0_b']),
        p['d1W'][:HD], p['d1W'][HD:], row(p['d1b']),
        row(p['dbn1_g'] * _BNS), row(p['dbn1_b']),
        Wd, gdt, P)
    return adjf.reshape(NG, NMAX, NMAX)
